# baseline (speedup 1.0000x reference)
import jax
import jax.numpy as jnp
from jax.experimental import pallas as pl

_N = 8192
_C = 384
_H = 12
_D = 32
_WN = 128
_M = 64
_QS = 4
_WSZ = 5
_LX = 2 * _WSZ * _QS
_CQS = 2 * _QS
_CW = 2
_LR = 2 * _CW * _CQS
_K = 3 * _LX + 3 * _LR
_SCALE = _D ** -0.5


_WPB = 2


def _win_kernel(feats_ref, nc_ref, wqkv_ref, bqkv_ref, wproj_ref, bproj_ref,
                wdq_ref, bdq_ref, wdkF_ref, bdkF_ref, vtab_ref, out_ref):
    for w in range(_WPB):
        _one_window(feats_ref[w * _M:(w + 1) * _M, :],
                    nc_ref[w * _M:(w + 1) * _M, :],
                    wqkv_ref, bqkv_ref, wproj_ref, bproj_ref,
                    wdq_ref, bdq_ref, wdkF_ref, bdkF_ref, vtab_ref,
                    out_ref, w)


def _one_window(x, nc, wqkv_ref, bqkv_ref, wproj_ref, bproj_ref,
                wdq_ref, bdq_ref, wdkF_ref, bdkF_ref, vtab_ref,
                out_ref, w):
    f32 = jnp.float32
    bf = jnp.bfloat16

    qkv = jnp.dot(x, wqkv_ref[...],
                  preferred_element_type=f32) + bqkv_ref[...]
    qkv = qkv.reshape(_M, 3, _H, _D)
    q = qkv[:, 0] * _SCALE
    k = qkv[:, 1]
    v = qkv[:, 2]

    xi = jnp.clip(jnp.floor(nc[:, 0:3] * _QS), 0, _WSZ * _QS - 1).astype(jnp.int32)
    ri = jnp.clip(jnp.floor(nc[:, 3:6] * _CQS), 0, _CW * _CQS - 1).astype(jnp.int32)
    xiT = xi.T
    riT = ri.T

    kk = jax.lax.broadcasted_iota(jnp.int32, (_M, _M, _K), 2).astype(bf)
    xib = xi.astype(bf)
    rib = ri.astype(bf)
    xibT = xib.T
    ribT = rib.T
    m = None
    for a in range(3):
        dxa = (xib[:, a:a + 1] - xibT[a:a + 1, :]
               + float(_LX // 2 + a * _LX))[:, :, None]
        dra = (rib[:, a:a + 1] - ribT[a:a + 1, :]
               + float(_LR // 2 + 3 * _LX + a * _LR))[:, :, None]
        e = jnp.logical_or(kk == dxa, kk == dra)
        m = e if m is None else jnp.logical_or(m, e)
    oh = m.astype(bf)

    vtab = vtab_ref[...]
    xb = x.astype(bf)

    dq = (jnp.dot(xb, wdq_ref[...], preferred_element_type=f32)
          + bdq_ref[...]).astype(bf).reshape(_M, _K, _H)
    dkF = (jnp.dot(xb, wdkF_ref[...], preferred_element_type=f32)
           + bdkF_ref[...]).astype(bf).reshape(_M, _K, _H)

    g = jnp.einsum('ikh,ijk->ijh', dq, oh, preferred_element_type=f32)
    p2 = jnp.einsum('jkh,jik->jih', dkF, oh, preferred_element_type=f32)
    g += p2.transpose(1, 0, 2)

    attn = jnp.einsum('ihd,jhd->ijh', q, k) + g
    p = jax.nn.softmax(attn, axis=1)
    pb = p.astype(bf)

    out = jnp.einsum('ijh,jhd->ihd', p, v)
    sv = jnp.einsum('ijh,ijk->ikh', pb, oh, preferred_element_type=f32).astype(bf)
    out += jnp.einsum('ikh,khd->ihd', sv, vtab, preferred_element_type=f32)

    out = out.reshape(_M, _C)
    out_ref[w * _M:(w + 1) * _M, :] = jnp.dot(
        out, wproj_ref[...], preferred_element_type=f32) + bproj_ref[...]


def kernel(feats, n_coords, w_qkv, b_qkv, w_proj, b_proj,
           q_xyz_table, k_xyz_table, v_xyz_table,
           q_rgb_table, k_rgb_table, v_rgb_table):
    bf = jnp.bfloat16

    fx = (-jnp.arange(_LX)) % _LX
    fr = (-jnp.arange(_LR)) % _LR
    ktab = jnp.concatenate(
        [k_xyz_table.reshape(3 * _LX, _H, _D),
         k_rgb_table.reshape(3 * _LR, _H, _D)], axis=0
    )
    qtabF = jnp.concatenate(
        [q_xyz_table[:, fx].reshape(3 * _LX, _H, _D),
         q_rgb_table[:, fr].reshape(3 * _LR, _H, _D)], axis=0
    )
    vtab = jnp.concatenate(
        [v_xyz_table.reshape(3 * _LX, _H, _D),
         v_rgb_table.reshape(3 * _LR, _H, _D)], axis=0
    ).astype(bf)

    wq = w_qkv[:, 0:_C].reshape(_C, _H, _D)
    bq = b_qkv[0:_C].reshape(_H, _D)
    wk = w_qkv[:, _C:2 * _C].reshape(_C, _H, _D)
    bk = b_qkv[_C:2 * _C].reshape(_H, _D)
    wdq = (jnp.einsum('chd,khd->ckh', wq, ktab) * _SCALE).reshape(_C, _K * _H).astype(bf)
    bdq = (jnp.einsum('hd,khd->kh', bq, ktab) * _SCALE).reshape(_K * _H)
    wdkF = jnp.einsum('chd,khd->ckh', wk, qtabF).reshape(_C, _K * _H).astype(bf)
    bdkF = jnp.einsum('hd,khd->kh', bk, qtabF).reshape(_K * _H)

    full = lambda shape: pl.BlockSpec(shape, lambda w: (0,) * len(shape))
    grid_spec = pl.GridSpec(
        grid=(_WN // _WPB,),
        in_specs=[
            pl.BlockSpec((_WPB * _M, _C), lambda w: (w, 0)),
            pl.BlockSpec((_WPB * _M, 6), lambda w: (w, 0)),
            full((_C, 3 * _C)),
            full((3 * _C,)),
            full((_C, _C)),
            full((_C,)),
            full((_C, _K * _H)),
            full((_K * _H,)),
            full((_C, _K * _H)),
            full((_K * _H,)),
            full((_K, _H, _D)),
        ],
        out_specs=pl.BlockSpec((_WPB * _M, _C), lambda w: (w, 0)),
    )
    return pl.pallas_call(
        _win_kernel,
        grid_spec=grid_spec,
        out_shape=jax.ShapeDtypeStruct((_N, _C), jnp.float32),
    )(feats, n_coords, w_qkv, b_qkv, w_proj, b_proj,
      wdq, bdq, wdkF, bdkF, vtab)


# speedup vs baseline: 1.0438x; 1.0438x over previous
import jax
import jax.numpy as jnp
from jax.experimental import pallas as pl

_N = 8192
_C = 384
_H = 12
_D = 32
_WN = 128
_M = 64
_QS = 4
_WSZ = 5
_LX = 2 * _WSZ * _QS
_CQS = 2 * _QS
_CW = 2
_LR = 2 * _CW * _CQS
_K = 3 * _LX + 3 * _LR
_SCALE = _D ** -0.5


def _win_kernel(feats_ref, nc_ref, wqkv_ref, bqkv_ref, wproj_ref, bproj_ref,
                wdq_ref, bdq_ref, wdkF_ref, bdkF_ref, vtab_ref, out_ref):
    f32 = jnp.float32
    bf = jnp.bfloat16

    x = feats_ref[...]
    qkv = jnp.dot(x, wqkv_ref[...],
                  preferred_element_type=f32) + bqkv_ref[...]
    qkv = qkv.reshape(_M, 3, _H, _D)
    q = qkv[:, 0] * _SCALE
    k = qkv[:, 1]
    v = qkv[:, 2]

    nc = nc_ref[...]
    xi = jnp.clip(jnp.floor(nc[:, 0:3] * _QS), 0, _WSZ * _QS - 1).astype(jnp.int32)
    ri = jnp.clip(jnp.floor(nc[:, 3:6] * _CQS), 0, _CW * _CQS - 1).astype(jnp.int32)
    xiT = xi.T
    riT = ri.T

    kk = jax.lax.broadcasted_iota(jnp.int32, (_M, _M, _K), 2).astype(bf)
    xib = xi.astype(bf)
    rib = ri.astype(bf)
    xibT = xib.T
    ribT = rib.T
    m = None
    for a in range(3):
        dxa = (xib[:, a:a + 1] - xibT[a:a + 1, :]
               + float(_LX // 2 + a * _LX))[:, :, None]
        dra = (rib[:, a:a + 1] - ribT[a:a + 1, :]
               + float(_LR // 2 + 3 * _LX + a * _LR))[:, :, None]
        e = jnp.logical_or(kk == dxa, kk == dra)
        m = e if m is None else jnp.logical_or(m, e)
    oh = m.astype(bf)

    vtab = vtab_ref[...]
    xb = x.astype(bf)

    dq = (jnp.dot(xb, wdq_ref[...], preferred_element_type=f32)
          + bdq_ref[...]).astype(bf).reshape(_M, _K, _H)
    dkF = (jnp.dot(xb, wdkF_ref[...], preferred_element_type=f32)
           + bdkF_ref[...]).astype(bf).reshape(_M, _K, _H)

    g = jnp.einsum('ikh,ijk->ijh', dq, oh, preferred_element_type=f32)
    p2 = jnp.einsum('jkh,jik->jih', dkF, oh, preferred_element_type=f32)
    g += p2.transpose(1, 0, 2)

    attn = jnp.einsum('ihd,jhd->ijh', q.astype(bf), k.astype(bf),
                  preferred_element_type=f32) + g
    p = jax.nn.softmax(attn, axis=1)
    pb = p.astype(bf)

    out = jnp.einsum('ijh,jhd->ihd', pb, v.astype(bf),
                 preferred_element_type=f32)
    sv = jnp.einsum('ijh,ijk->ikh', pb, oh, preferred_element_type=f32).astype(bf)
    out += jnp.einsum('ikh,khd->ihd', sv, vtab, preferred_element_type=f32)

    out = out.reshape(_M, _C)
    out_ref[...] = jnp.dot(out, wproj_ref[...],
                           preferred_element_type=f32) + bproj_ref[...]


def kernel(feats, n_coords, w_qkv, b_qkv, w_proj, b_proj,
           q_xyz_table, k_xyz_table, v_xyz_table,
           q_rgb_table, k_rgb_table, v_rgb_table):
    bf = jnp.bfloat16

    fx = (-jnp.arange(_LX)) % _LX
    fr = (-jnp.arange(_LR)) % _LR
    ktab = jnp.concatenate(
        [k_xyz_table.reshape(3 * _LX, _H, _D),
         k_rgb_table.reshape(3 * _LR, _H, _D)], axis=0
    )
    qtabF = jnp.concatenate(
        [q_xyz_table[:, fx].reshape(3 * _LX, _H, _D),
         q_rgb_table[:, fr].reshape(3 * _LR, _H, _D)], axis=0
    )
    vtab = jnp.concatenate(
        [v_xyz_table.reshape(3 * _LX, _H, _D),
         v_rgb_table.reshape(3 * _LR, _H, _D)], axis=0
    ).astype(bf)

    wq = w_qkv[:, 0:_C].reshape(_C, _H, _D)
    bq = b_qkv[0:_C].reshape(_H, _D)
    wk = w_qkv[:, _C:2 * _C].reshape(_C, _H, _D)
    bk = b_qkv[_C:2 * _C].reshape(_H, _D)
    wdq = (jnp.einsum('chd,khd->ckh', wq, ktab) * _SCALE).reshape(_C, _K * _H).astype(bf)
    bdq = (jnp.einsum('hd,khd->kh', bq, ktab) * _SCALE).reshape(_K * _H)
    wdkF = jnp.einsum('chd,khd->ckh', wk, qtabF).reshape(_C, _K * _H).astype(bf)
    bdkF = jnp.einsum('hd,khd->kh', bk, qtabF).reshape(_K * _H)

    full = lambda shape: pl.BlockSpec(shape, lambda w: (0,) * len(shape))
    grid_spec = pl.GridSpec(
        grid=(_WN,),
        in_specs=[
            pl.BlockSpec((_M, _C), lambda w: (w, 0)),
            pl.BlockSpec((_M, 6), lambda w: (w, 0)),
            full((_C, 3 * _C)),
            full((3 * _C,)),
            full((_C, _C)),
            full((_C,)),
            full((_C, _K * _H)),
            full((_K * _H,)),
            full((_C, _K * _H)),
            full((_K * _H,)),
            full((_K, _H, _D)),
        ],
        out_specs=pl.BlockSpec((_M, _C), lambda w: (w, 0)),
    )
    return pl.pallas_call(
        _win_kernel,
        grid_spec=grid_spec,
        out_shape=jax.ShapeDtypeStruct((_N, _C), jnp.float32),
    )(feats, n_coords, w_qkv, b_qkv, w_proj, b_proj,
      wdq, bdq, wdkF, bdkF, vtab)
